# Initial kernel scaffold; baseline (speedup 1.0000x reference)
import jax
import jax.numpy as jnp
from jax.experimental import pallas as pl


def kernel(x, adj_a, adj_b, W_gcn, b_gcn, W_na, a_na, Wa, ba, Wb, bb):
    raise NotImplementedError("write your pallas kernel here")


# speedup vs baseline: 1.2077x; 1.2077x over previous
import functools

import jax
import jax.numpy as jnp
from jax.experimental import pallas as pl
import jax.experimental.pallas.tpu as pltpu

N = 1024
IN = 128
OUT = 128


def _fused_body(x_ref, aa_ref, ab_ref, wg_ref, bg_ref, wn_ref, an_ref,
                wa_ref, ba_ref, wb_ref, bb_ref, out_ref):
    f32 = jnp.float32
    x = x_ref[...]
    aa = aa_ref[...]
    ab = ab_ref[...]

    xh = jnp.dot(x, wn_ref[...], preferred_element_type=f32)
    a_vec = an_ref[...]
    s_src = jnp.dot(xh, a_vec[0, :OUT].reshape(OUT, 1),
                    preferred_element_type=f32)
    s_dst = jnp.dot(xh, a_vec[0, OUT:].reshape(OUT, 1),
                    preferred_element_type=f32)
    scores = s_src + s_dst.reshape(1, N)
    lrelu = jnp.where(scores >= 0, scores, 0.01 * scores)
    dense = jnp.where(aa != 0, jnp.exp(-lrelu), 0.0)
    r = jnp.sum(dense, axis=1)
    inv = 1.0 / (r + 1e-05)
    x_a = jnp.dot(dense, xh * inv[:, None], preferred_element_type=f32)

    ts_a = jnp.dot(aa, x, preferred_element_type=f32)
    ts_b = jnp.dot(ab, x, preferred_element_type=f32)
    wa = wa_ref[...]
    wb = wb_ref[...]
    attn_a = jax.nn.sigmoid(
        jnp.dot(ts_a, wa[0, :IN].reshape(IN, 1), preferred_element_type=f32)
        + jnp.dot(x, wa[0, IN:].reshape(IN, 1), preferred_element_type=f32)
        + ba_ref[0])
    attn_b = jax.nn.sigmoid(
        jnp.dot(ts_b, wb[0, :IN].reshape(IN, 1), preferred_element_type=f32)
        + jnp.dot(x, wb[0, IN:].reshape(IN, 1), preferred_element_type=f32)
        + bb_ref[0])

    xg = jnp.dot(x, wg_ref[...], preferred_element_type=f32)
    x_b = jnp.dot(ab, xg, preferred_element_type=f32) + bg_ref[...]

    out_ref[...] = jax.nn.sigmoid(attn_a * x_a + attn_b * x_b)


@jax.jit
def kernel(x, adj_a, adj_b, W_gcn, b_gcn, W_na, a_na, Wa, ba, Wb, bb):
    return pl.pallas_call(
        _fused_body,
        out_shape=jax.ShapeDtypeStruct((N, OUT), jnp.float32),
    )(x, adj_a, adj_b, W_gcn, b_gcn.reshape(1, OUT), W_na, a_na,
      Wa, ba, Wb, bb)


# speedup vs baseline: 1.4324x; 1.1860x over previous
import functools

import jax
import jax.numpy as jnp
from jax.experimental import pallas as pl
import jax.experimental.pallas.tpu as pltpu

N = 1024
IN = 128
OUT = 128


def _fused_body(x_ref, aa_ref, ab_ref, wg_ref, bg_ref, wn_ref, an_ref,
                wa_ref, ba_ref, wb_ref, bb_ref, out_ref):
    f32 = jnp.float32
    x = x_ref[...]
    aa = aa_ref[...]
    ab = ab_ref[...]

    xh = jnp.dot(x, wn_ref[...], preferred_element_type=f32)
    a_vec = an_ref[...]
    s_src = jnp.dot(xh, a_vec[0, :OUT].reshape(OUT, 1),
                    preferred_element_type=f32)
    s_dst = jnp.dot(xh, a_vec[0, OUT:].reshape(OUT, 1),
                    preferred_element_type=f32)
    scores = s_src + s_dst.reshape(1, N)
    lrelu = jnp.where(scores >= 0, scores, 0.01 * scores)
    dense = jnp.where(aa != 0, jnp.exp(-lrelu), 0.0)
    r = jnp.sum(dense, axis=1)
    inv = 1.0 / (r + 1e-05)
    x_a = jnp.dot(dense, xh * inv[:, None], preferred_element_type=f32)

    wa = wa_ref[...]
    wb = wb_ref[...]
    v_a = jnp.dot(x, wa[0, :IN].reshape(IN, 1), preferred_element_type=f32)
    v_b = jnp.dot(x, wb[0, :IN].reshape(IN, 1), preferred_element_type=f32)
    m_a = jnp.sum(aa * v_a.reshape(1, N), axis=1, keepdims=True)
    m_b = jnp.sum(ab * v_b.reshape(1, N), axis=1, keepdims=True)
    attn_a = jax.nn.sigmoid(
        m_a + jnp.dot(x, wa[0, IN:].reshape(IN, 1), preferred_element_type=f32)
        + ba_ref[0])
    attn_b = jax.nn.sigmoid(
        m_b + jnp.dot(x, wb[0, IN:].reshape(IN, 1), preferred_element_type=f32)
        + bb_ref[0])

    xg = jnp.dot(x, wg_ref[...], preferred_element_type=f32)
    x_b = jnp.dot(ab, xg, preferred_element_type=f32) + bg_ref[...]

    out_ref[...] = jax.nn.sigmoid(attn_a * x_a + attn_b * x_b)


@jax.jit
def kernel(x, adj_a, adj_b, W_gcn, b_gcn, W_na, a_na, Wa, ba, Wb, bb):
    return pl.pallas_call(
        _fused_body,
        out_shape=jax.ShapeDtypeStruct((N, OUT), jnp.float32),
    )(x, adj_a, adj_b, W_gcn, b_gcn.reshape(1, OUT), W_na, a_na,
      Wa, ba, Wb, bb)
